# baseline (speedup 1.0000x reference)
import functools

import jax
import jax.numpy as jnp
from jax import lax
from jax.experimental import pallas as pl
from jax.experimental.pallas import tpu as pltpu
from jax.experimental.pallas import tpu_sc as plsc

N = 10000
E = 160000
H2 = 64
LANES = 16
NCORES = 2
NTILES = 16
RPT = 624
RTAIL = N - RPT * NTILES
E_PER_TILE = E // NTILES
EB = 128
NB = 79
PT_FULL = NB * EB
NB_S = 41
PT_SPLIT = NB_S * EB
NDUMP = 16


def _mm_body_2d(a_ref, w_ref, o_ref, *, relu):
    a = a_ref[...]
    if relu:
        a = jnp.maximum(a, 0.0)
    o_ref[...] = jnp.dot(a, w_ref[0], preferred_element_type=jnp.float32)


def _mm_body_3d(a_ref, w_ref, o_ref, *, relu, ci, cw_in):
    acc = None
    for i in range(ci):
        a = a_ref[i]
        if relu:
            a = jnp.maximum(a, 0.0)
        p = jnp.dot(a, w_ref[0, i * cw_in:(i + 1) * cw_in, :],
                    preferred_element_type=jnp.float32)
        acc = p if acc is None else acc + p
    o_ref[...] = acc


def _mm(a, w, relu=False, cw_out=128, rows=1000):
    k, fo = w.shape
    co = fo // cw_out
    if a.ndim == 2:
        n = a.shape[0]
        body = functools.partial(_mm_body_2d, relu=relu)
        a_spec = pl.BlockSpec((rows, k), lambda c, r: (r, 0))
    else:
        ci, n, cw_in = a.shape
        body = functools.partial(_mm_body_3d, relu=relu, ci=ci, cw_in=cw_in)
        a_spec = pl.BlockSpec((ci, rows, cw_in), lambda c, r: (0, r, 0))
    nr = n // rows
    w3 = w.reshape(k, co, cw_out).transpose(1, 0, 2)
    return pl.pallas_call(
        body,
        grid=(co, nr),
        in_specs=[
            a_spec,
            pl.BlockSpec((1, k, cw_out), lambda c, r: (c, 0, 0)),
        ],
        out_specs=pl.BlockSpec((rows, cw_out), lambda c, r: (c * nr + r, 0)),
        out_shape=jax.ShapeDtypeStruct((co * n, cw_out), jnp.float32),
    )(a, w3)


def _mlz_body(p0_ref, p1_ref, eps_ref, mu_ref, lv_ref, z_ref, zp_ref):
    s = p0_ref[...] + p1_ref[...]
    mu = s[:, :H2]
    lv = s[:, H2:]
    mu_ref[...] = mu
    lv_ref[...] = lv
    z = eps_ref[...] * jnp.exp(lv) + mu
    z_ref[...] = z
    zp_ref[...] = jnp.concatenate([z, jnp.zeros_like(z)], axis=1)


def _mlz(agg23p, eps, rows=1000):
    nr = N // rows
    ospec = pl.BlockSpec((rows, H2), lambda r: (r, 0))
    oshape = jax.ShapeDtypeStruct((N, H2), jnp.float32)
    pspec = pl.BlockSpec((rows, 2 * H2), lambda r: (r, 0))
    return pl.pallas_call(
        _mlz_body,
        grid=(nr,),
        in_specs=[
            pspec,
            pl.BlockSpec((rows, 2 * H2), lambda r: (r + nr, 0)),
            ospec,
        ],
        out_specs=[ospec, ospec, ospec, pspec],
        out_shape=[oshape, oshape, oshape,
                   jax.ShapeDtypeStruct((N, 2 * H2), jnp.float32)],
    )(agg23p, agg23p, eps)


def _mm2_body(p0_ref, p1_ref, w1_ref, w2_ref, o_ref):
    az = (p0_ref[...] + p1_ref[...])[:, :H2]
    hd = jnp.maximum(jnp.dot(az, w1_ref[...],
                             preferred_element_type=jnp.float32), 0.0)
    o_ref[...] = jnp.dot(hd, w2_ref[0], preferred_element_type=jnp.float32)


def _mm2(azp, w1, w2, rows=1000):
    nr = N // rows
    w23 = w2.reshape(w2.shape[0], 2, 128).transpose(1, 0, 2)
    return pl.pallas_call(
        _mm2_body,
        grid=(2, nr),
        in_specs=[
            pl.BlockSpec((rows, 2 * H2), lambda c, r: (r, 0)),
            pl.BlockSpec((rows, 2 * H2), lambda c, r: (r + nr, 0)),
            pl.BlockSpec((H2, w1.shape[1]), lambda c, r: (0, 0)),
            pl.BlockSpec((1, w2.shape[0], 128), lambda c, r: (c, 0, 0)),
        ],
        out_specs=pl.BlockSpec((rows, 128), lambda c, r: (c * nr + r, 0)),
        out_shape=jax.ShapeDtypeStruct((2 * N, 128), jnp.float32),
    )(azp, azp, w1, w23)


def _zzt_body(a_ref, b_ref, o_ref):
    o_ref[...] = lax.dot_general(
        a_ref[...], b_ref[...], (((1,), (1,)), ((), ())),
        preferred_element_type=jnp.float32)


def _zzt(z, rows=400):
    nr = N // rows
    return pl.pallas_call(
        _zzt_body,
        grid=(nr,),
        in_specs=[
            pl.BlockSpec((rows, H2), lambda i: (i, 0)),
            pl.BlockSpec((N, H2), lambda i: (0, 0)),
        ],
        out_specs=pl.BlockSpec((rows, N), lambda i: (i, 0)),
        out_shape=jax.ShapeDtypeStruct((N, N), jnp.float32),
    )(z, z)


def _spmm(sup_flat, srcz, dst, split=False):
    mesh = plsc.VectorSubcoreMesh(core_axis_name="c", subcore_axis_name="s")
    zeros = jnp.zeros((N + NDUMP, 128), jnp.float32)
    eb = EB
    nb = NB_S if split else NB
    assert nb % 2 == 1

    @functools.partial(
        pl.kernel,
        mesh=mesh,
        out_type=jax.ShapeDtypeStruct((NCORES * N, 128), jnp.float32),
        scratch_types=[
            pltpu.VMEM((2, eb), jnp.int32),
            pltpu.VMEM((2, eb), jnp.int32),
            pltpu.VMEM((2, eb, 128), jnp.float32),
            pltpu.VMEM_SHARED((N + NDUMP, 128), jnp.float32),
            pltpu.SemaphoreType.DMA,
            pltpu.SemaphoreType.DMA,
            pltpu.SemaphoreType.DMA,
            pltpu.SemaphoreType.DMA,
        ],
    )
    def k(sup_hbm, src_hbm, dst_hbm, zer_hbm, out_hbm,
          src_v, dst_v, rows_v, acc, si0, si1, sg0, sg1):
        cid = lax.axis_index("c")
        sid = lax.axis_index("s")
        row0 = pl.multiple_of(sid * RPT, 8)
        if split:
            sbase0 = cid * (NTILES * PT_SPLIT) + sid * PT_SPLIT
            dbase0 = sbase0
        else:
            sbase0 = cid * (NTILES * PT_FULL) + sid * PT_FULL
            dbase0 = sid * PT_FULL
        last = sid == NTILES - 1
        sem_i = (si0, si1)
        sem_g = (sg0, sg1)

        pltpu.sync_copy(zer_hbm.at[pl.ds(row0, RPT)], acc.at[pl.ds(row0, RPT)])

        @pl.when(last)
        def _():
            pltpu.sync_copy(zer_hbm.at[pl.ds(RPT * NTILES, RTAIL + NDUMP)],
                            acc.at[pl.ds(RPT * NTILES, RTAIL + NDUMP)])

        plsc.subcore_barrier()

        def idx_start(b, j):
            sb = pl.multiple_of(sbase0 + b * eb, 8)
            db = pl.multiple_of(dbase0 + b * eb, 8)
            pltpu.make_async_copy(
                src_hbm.at[pl.ds(sb, eb)], src_v.at[j], sem_i[j]).start()
            pltpu.make_async_copy(
                dst_hbm.at[pl.ds(db, eb)], dst_v.at[j], sem_i[j]).start()

        def idx_wait(j):
            pltpu.make_async_copy(
                src_hbm.at[pl.ds(0, eb)], src_v.at[j], sem_i[j]).wait()
            pltpu.make_async_copy(
                dst_hbm.at[pl.ds(0, eb)], dst_v.at[j], sem_i[j]).wait()

        def gather_start(j):
            pltpu.make_async_copy(
                sup_hbm.at[src_v.at[j]], rows_v.at[j], sem_g[j]).start()

        def gather_wait(j):
            pltpu.make_async_copy(
                sup_hbm.at[src_v.at[j]], rows_v.at[j], sem_g[j]).wait()

        def scatter(j):
            pltpu.sync_copy(rows_v.at[j], acc.at[dst_v.at[j]], add=True)

        idx_start(0, 0)
        idx_start(1, 1)
        idx_wait(0)
        gather_start(0)

        def pair(g, carry):
            b = 2 * g
            idx_wait(1)
            gather_start(1)
            gather_wait(0)
            scatter(0)

            @pl.when(b + 2 < nb)
            def _():
                idx_start(b + 2, 0)
                idx_wait(0)
                gather_start(0)

            gather_wait(1)
            scatter(1)

            @pl.when(b + 3 < nb)
            def _():
                idx_start(b + 3, 1)

            return carry

        lax.fori_loop(0, nb // 2, pair, 0)
        gather_wait(0)
        scatter(0)
        plsc.subcore_barrier()

        obase = pl.multiple_of(cid * N + row0, 8)
        pltpu.sync_copy(acc.at[pl.ds(row0, RPT)], out_hbm.at[pl.ds(obase, RPT)])

        @pl.when(last)
        def _():
            pltpu.sync_copy(
                acc.at[pl.ds(RPT * NTILES, RTAIL)],
                out_hbm.at[pl.ds(pl.multiple_of(cid * N + RPT * NTILES, 8),
                                 RTAIL)])

    return k(sup_flat, srcz, dst, zeros)


def kernel(x, edge_index, W1, W2, W3, Wd1, Wd2):
    src = edge_index[0]
    dst = edge_index[1]

    def pad_seg(a, old, new, off):
        a2 = a.reshape(-1, old)
        fill = jnp.arange(new - old, dtype=jnp.int32) % NDUMP + off
        padv = jnp.broadcast_to(fill, (a2.shape[0], new - old))
        return jnp.concatenate([a2, padv], axis=1).reshape(-1)

    sseg = pad_seg(src, E_PER_TILE, PT_FULL, 0)
    srcz = jnp.concatenate([sseg, sseg + N])
    dstz = pad_seg(dst, E_PER_TILE, PT_FULL, N)
    srcp = pad_seg(src, E // 32, PT_SPLIT, 0)
    dstp = pad_seg(dst, E // 32, PT_SPLIT, N)
    W23 = jnp.concatenate([W2, W3], axis=1)
    eps = jax.random.normal(jax.random.key(1), (N, H2), dtype=jnp.float32)

    sup1 = _mm(x, W1)
    agg1 = _spmm(sup1, srcz, dstz)
    sup23 = _mm(agg1.reshape(2, N, 128), W23, relu=True)
    agg23p = _spmm(sup23, srcp, dstp, split=True)
    mu, logvar, z, zpad = _mlz(agg23p, eps)

    azp = _spmm(zpad, srcp, dstp, split=True)
    supd2 = _mm2(azp, Wd1, Wd2)
    z_t, _ = jax.lax.optimization_barrier((z, supd2))
    xr = _spmm(supd2, srcz, dstz)
    recon_adj = _zzt(z_t)
    x_rec = xr.reshape(2, N, 128).transpose(1, 0, 2).reshape(N, 256)

    return (recon_adj, mu, logvar, z, x_rec)


# speedup vs baseline: 1.1401x; 1.1401x over previous
import functools

import jax
import jax.numpy as jnp
from jax import lax
from jax.experimental import pallas as pl
from jax.experimental.pallas import tpu as pltpu
from jax.experimental.pallas import tpu_sc as plsc

N = 10000
E = 160000
H2 = 64
LANES = 16
NCORES = 2
NTILES = 16
RPT = 624
RTAIL = N - RPT * NTILES
E_PER_TILE = E // NTILES
EB = 128
NB = 79
PT_FULL = NB * EB
NB_S = 41
PT_SPLIT = NB_S * EB
NDUMP = 16


_MM_PARAMS = pltpu.CompilerParams(vmem_limit_bytes=16 * 1024 * 1024)
_ZZT_PARAMS = pltpu.CompilerParams(vmem_limit_bytes=40 * 1024 * 1024)
_SC_PARAMS = pltpu.CompilerParams(vmem_limit_bytes=4 * 1024 * 1024)


def _mm_body_2d(a_ref, w_ref, o_ref, *, relu):
    a = a_ref[...]
    if relu:
        a = jnp.maximum(a, 0.0)
    o_ref[...] = jnp.dot(a, w_ref[0], preferred_element_type=jnp.float32)


def _mm_body_3d(a_ref, w_ref, o_ref, *, relu, ci, cw_in):
    acc = None
    for i in range(ci):
        a = a_ref[i]
        if relu:
            a = jnp.maximum(a, 0.0)
        p = jnp.dot(a, w_ref[0, i * cw_in:(i + 1) * cw_in, :],
                    preferred_element_type=jnp.float32)
        acc = p if acc is None else acc + p
    o_ref[...] = acc


def _mm(a, w, relu=False, cw_out=128, rows=1000):
    k, fo = w.shape
    co = fo // cw_out
    if a.ndim == 2:
        n = a.shape[0]
        body = functools.partial(_mm_body_2d, relu=relu)
        a_spec = pl.BlockSpec((rows, k), lambda c, r: (r, 0))
    else:
        ci, n, cw_in = a.shape
        body = functools.partial(_mm_body_3d, relu=relu, ci=ci, cw_in=cw_in)
        a_spec = pl.BlockSpec((ci, rows, cw_in), lambda c, r: (0, r, 0))
    nr = n // rows
    w3 = w.reshape(k, co, cw_out).transpose(1, 0, 2)
    return pl.pallas_call(
        body,
        grid=(co, nr),
        in_specs=[
            a_spec,
            pl.BlockSpec((1, k, cw_out), lambda c, r: (c, 0, 0)),
        ],
        out_specs=pl.BlockSpec((rows, cw_out), lambda c, r: (c * nr + r, 0)),
        out_shape=jax.ShapeDtypeStruct((co * n, cw_out), jnp.float32),
        compiler_params=_MM_PARAMS,
    )(a, w3)


def _mlz_body(p0_ref, p1_ref, eps_ref, mu_ref, lv_ref, z_ref, zp_ref):
    s = p0_ref[...] + p1_ref[...]
    mu = s[:, :H2]
    lv = s[:, H2:]
    mu_ref[...] = mu
    lv_ref[...] = lv
    z = eps_ref[...] * jnp.exp(lv) + mu
    z_ref[...] = z
    zp_ref[...] = jnp.concatenate([z, jnp.zeros_like(z)], axis=1)


def _mlz(agg23p, eps, rows=1000):
    nr = N // rows
    ospec = pl.BlockSpec((rows, H2), lambda r: (r, 0))
    oshape = jax.ShapeDtypeStruct((N, H2), jnp.float32)
    pspec = pl.BlockSpec((rows, 2 * H2), lambda r: (r, 0))
    return pl.pallas_call(
        _mlz_body,
        grid=(nr,),
        in_specs=[
            pspec,
            pl.BlockSpec((rows, 2 * H2), lambda r: (r + nr, 0)),
            ospec,
        ],
        out_specs=[ospec, ospec, ospec, pspec],
        out_shape=[oshape, oshape, oshape,
                   jax.ShapeDtypeStruct((N, 2 * H2), jnp.float32)],
        compiler_params=_MM_PARAMS,
    )(agg23p, agg23p, eps)


def _mm2_body(p0_ref, p1_ref, w1_ref, w2_ref, o_ref):
    az = (p0_ref[...] + p1_ref[...])[:, :H2]
    hd = jnp.maximum(jnp.dot(az, w1_ref[...],
                             preferred_element_type=jnp.float32), 0.0)
    o_ref[...] = jnp.dot(hd, w2_ref[0], preferred_element_type=jnp.float32)


def _mm2(azp, w1, w2, rows=1000):
    nr = N // rows
    w23 = w2.reshape(w2.shape[0], 2, 128).transpose(1, 0, 2)
    return pl.pallas_call(
        _mm2_body,
        grid=(2, nr),
        in_specs=[
            pl.BlockSpec((rows, 2 * H2), lambda c, r: (r, 0)),
            pl.BlockSpec((rows, 2 * H2), lambda c, r: (r + nr, 0)),
            pl.BlockSpec((H2, w1.shape[1]), lambda c, r: (0, 0)),
            pl.BlockSpec((1, w2.shape[0], 128), lambda c, r: (c, 0, 0)),
        ],
        out_specs=pl.BlockSpec((rows, 128), lambda c, r: (c * nr + r, 0)),
        out_shape=jax.ShapeDtypeStruct((2 * N, 128), jnp.float32),
        compiler_params=_MM_PARAMS,
    )(azp, azp, w1, w23)


def _zzt_half_body(a_ref, b_ref, prev_ref, o_ref):
    del prev_ref
    o_ref[...] = lax.dot_general(
        a_ref[...], b_ref[...], (((1,), (1,)), ((), ())),
        preferred_element_type=jnp.float32)


def _zzt_half(z, first, nblk, prev, rows=200):
    nr = nblk
    in_specs = [
        pl.BlockSpec((rows, H2), lambda i, f=first: (i + f, 0)),
        pl.BlockSpec((N, H2), lambda i: (0, 0)),
    ]
    args = (z, z)
    aliases = {}
    body = _zzt_half_body
    if prev is None:
        def body(a_ref, b_ref, o_ref):
            return _zzt_half_body(a_ref, b_ref, None, o_ref)
    else:
        in_specs.append(pl.BlockSpec(memory_space=pl.ANY))
        args = (z, z, prev)
        aliases = {2: 0}
    return pl.pallas_call(
        body,
        grid=(nr,),
        in_specs=in_specs,
        out_specs=pl.BlockSpec((rows, N), lambda i, f=first: (i + f, 0)),
        out_shape=jax.ShapeDtypeStruct((N, N), jnp.float32),
        input_output_aliases=aliases,
        cost_estimate=pl.CostEstimate(
            flops=2 * nblk * rows * N * H2, transcendentals=0,
            bytes_accessed=4 * (nblk * rows * N + 2 * N * H2)),
        compiler_params=_ZZT_PARAMS,
    )(*args)


def _spmm(sup_flat, srcz, dst, split=False, out2d=False):
    mesh = plsc.VectorSubcoreMesh(core_axis_name="c", subcore_axis_name="s")
    zeros = jnp.zeros((N + NDUMP, 128), jnp.float32)
    eb = EB
    nb = NB_S if split else NB
    assert nb % 2 == 1

    nedge = NCORES * NTILES * (PT_SPLIT if split else PT_FULL)
    cost = pl.CostEstimate(
        flops=nedge * 128,
        transcendentals=0,
        bytes_accessed=nedge * 2 * 512 + NCORES * N * 512,
    )

    @functools.partial(
        pl.kernel,
        mesh=mesh,
        cost_estimate=cost,
        compiler_params=_SC_PARAMS,
        out_type=jax.ShapeDtypeStruct(
            (N, 256) if out2d else (NCORES * N, 128), jnp.float32),
        scratch_types=[
            pltpu.VMEM((2, eb), jnp.int32),
            pltpu.VMEM((2, eb), jnp.int32),
            pltpu.VMEM((2, eb, 128), jnp.float32),
            pltpu.VMEM_SHARED((N + NDUMP, 128), jnp.float32),
            pltpu.SemaphoreType.DMA,
            pltpu.SemaphoreType.DMA,
            pltpu.SemaphoreType.DMA,
            pltpu.SemaphoreType.DMA,
        ],
    )
    def k(sup_hbm, src_hbm, dst_hbm, zer_hbm, out_hbm,
          src_v, dst_v, rows_v, acc, si0, si1, sg0, sg1):
        cid = lax.axis_index("c")
        sid = lax.axis_index("s")
        row0 = pl.multiple_of(sid * RPT, 8)
        if split:
            sbase0 = cid * (NTILES * PT_SPLIT) + sid * PT_SPLIT
            dbase0 = sbase0
        else:
            sbase0 = cid * (NTILES * PT_FULL) + sid * PT_FULL
            dbase0 = sid * PT_FULL
        last = sid == NTILES - 1
        sem_i = (si0, si1)
        sem_g = (sg0, sg1)

        pltpu.sync_copy(zer_hbm.at[pl.ds(row0, RPT)], acc.at[pl.ds(row0, RPT)])

        @pl.when(last)
        def _():
            pltpu.sync_copy(zer_hbm.at[pl.ds(RPT * NTILES, RTAIL + NDUMP)],
                            acc.at[pl.ds(RPT * NTILES, RTAIL + NDUMP)])

        plsc.subcore_barrier()

        def idx_start(b, j):
            sb = pl.multiple_of(sbase0 + b * eb, 8)
            db = pl.multiple_of(dbase0 + b * eb, 8)
            pltpu.make_async_copy(
                src_hbm.at[pl.ds(sb, eb)], src_v.at[j], sem_i[j]).start()
            pltpu.make_async_copy(
                dst_hbm.at[pl.ds(db, eb)], dst_v.at[j], sem_i[j]).start()

        def idx_wait(j):
            pltpu.make_async_copy(
                src_hbm.at[pl.ds(0, eb)], src_v.at[j], sem_i[j]).wait()
            pltpu.make_async_copy(
                dst_hbm.at[pl.ds(0, eb)], dst_v.at[j], sem_i[j]).wait()

        def gather_start(j):
            pltpu.make_async_copy(
                sup_hbm.at[src_v.at[j]], rows_v.at[j], sem_g[j]).start()

        def gather_wait(j):
            pltpu.make_async_copy(
                sup_hbm.at[src_v.at[j]], rows_v.at[j], sem_g[j]).wait()

        def scatter(j):
            pltpu.sync_copy(rows_v.at[j], acc.at[dst_v.at[j]], add=True)

        idx_start(0, 0)
        idx_start(1, 1)
        idx_wait(0)
        gather_start(0)

        def pair(g, carry):
            b = 2 * g
            idx_wait(1)
            gather_start(1)
            gather_wait(0)
            scatter(0)

            @pl.when(b + 2 < nb)
            def _():
                idx_start(b + 2, 0)
                idx_wait(0)
                gather_start(0)

            gather_wait(1)
            scatter(1)

            @pl.when(b + 3 < nb)
            def _():
                idx_start(b + 3, 1)

            return carry

        lax.fori_loop(0, nb // 2, pair, 0)
        gather_wait(0)
        scatter(0)
        plsc.subcore_barrier()

        if out2d:
            col = pl.multiple_of(cid * 128, 128)
            pltpu.sync_copy(acc.at[pl.ds(row0, RPT)],
                            out_hbm.at[pl.ds(row0, RPT), pl.ds(col, 128)])

            @pl.when(last)
            def _():
                pltpu.sync_copy(
                    acc.at[pl.ds(RPT * NTILES, RTAIL)],
                    out_hbm.at[pl.ds(RPT * NTILES, RTAIL), pl.ds(col, 128)])
        else:
            obase = pl.multiple_of(cid * N + row0, 8)
            pltpu.sync_copy(acc.at[pl.ds(row0, RPT)],
                            out_hbm.at[pl.ds(obase, RPT)])

            @pl.when(last)
            def _():
                pltpu.sync_copy(
                    acc.at[pl.ds(RPT * NTILES, RTAIL)],
                    out_hbm.at[pl.ds(pl.multiple_of(cid * N + RPT * NTILES, 8),
                                     RTAIL)])

    return k(sup_flat, srcz, dst, zeros)


def kernel(x, edge_index, W1, W2, W3, Wd1, Wd2):
    src = edge_index[0]
    dst = edge_index[1]

    def pad_seg(a, old, new, off):
        a2 = a.reshape(-1, old)
        fill = jnp.arange(new - old, dtype=jnp.int32) % NDUMP + off
        padv = jnp.broadcast_to(fill, (a2.shape[0], new - old))
        return jnp.concatenate([a2, padv], axis=1).reshape(-1)

    sseg = pad_seg(src, E_PER_TILE, PT_FULL, 0)
    srcz = jnp.concatenate([sseg, sseg + N])
    dstz = pad_seg(dst, E_PER_TILE, PT_FULL, N)
    srcp = pad_seg(src, E // 32, PT_SPLIT, 0)
    dstp = pad_seg(dst, E // 32, PT_SPLIT, N)
    W23 = jnp.concatenate([W2, W3], axis=1)
    eps = jax.random.normal(jax.random.key(1), (N, H2), dtype=jnp.float32)

    sup1 = _mm(x, W1)
    agg1 = _spmm(sup1, srcz, dstz)
    sup23 = _mm(agg1.reshape(2, N, 128), W23, relu=True)
    agg23p = _spmm(sup23, srcp, dstp, split=True)
    mu, logvar, z, zpad = _mlz(agg23p, eps)

    azp = _spmm(zpad, srcp, dstp, split=True)
    ra0 = _zzt_half(z, 0, 18, None)
    azp_t, ra0 = jax.lax.optimization_barrier((azp, ra0))
    supd2 = _mm2(azp_t, Wd1, Wd2)
    x_rec = _spmm(supd2, srcz, dstz, out2d=True)
    recon_adj = _zzt_half(z, 18, 32, ra0)

    return (recon_adj, mu, logvar, z, x_rec)
